# baseline (speedup 1.0000x reference)
import functools

import jax
import jax.numpy as jnp
from jax import lax
from jax.experimental import pallas as pl
from jax.experimental.pallas import tpu as pltpu
from jax.experimental.pallas import tpu_sc as plsc

B = 128
N = 32768
K = 1639
QA = N - K + 1
NB = 256
NW = 32
ROWS = B // NW
INT_MIN = -(2**31)
XCAP = 16384
RH = 16448


def _scan_p1(hist, qa, qb):
    iota = lax.iota(jnp.int32, 16)
    z = jnp.zeros((16,), jnp.int32)

    def body(g, carry):
        run, bin_a, base_a, cnt_a, bin_b, cnt_b = carry
        tot = hist[pl.ds(g * 16, 16)]
        hist[pl.ds(g * 16, 16)] = z
        for l in range(1, 16):
            tot = tot + hist[pl.ds(l * NB + g * 16, 16)]
            hist[pl.ds(l * NB + g * 16, 16)] = z
        cum = run + plsc.cumsum(tot)
        prev = cum - tot
        ids = g * 16 + iota
        ma = (prev < qa) & (cum >= qa)
        mb = (prev < qb) & (cum >= qb)
        bin_a = bin_a + jnp.sum(jnp.where(ma, ids, 0))
        base_a = base_a + jnp.sum(jnp.where(ma, prev, 0))
        cnt_a = cnt_a + jnp.sum(jnp.where(ma, tot, 0))
        bin_b = bin_b + jnp.sum(jnp.where(mb, ids, 0))
        cnt_b = cnt_b + jnp.sum(jnp.where(mb, tot, 0))
        return run + jnp.sum(tot), bin_a, base_a, cnt_a, bin_b, cnt_b

    zi = jnp.int32(0)
    _, bin_a, base_a, cnt_a, bin_b, cnt_b = lax.fori_loop(
        0, NB // 16, body, (zi, zi, zi, zi, zi, zi))
    return bin_a, base_a, cnt_a, bin_b, cnt_b


def _scan1z(hist, q, off):
    iota = lax.iota(jnp.int32, 16)
    z = jnp.zeros((16,), jnp.int32)

    def body(g, carry):
        run, bin_, base = carry
        tot = hist[pl.ds(off + g * 16, 16)]
        hist[pl.ds(off + g * 16, 16)] = z
        for l in range(1, 16):
            tot = tot + hist[pl.ds(off + l * NB + g * 16, 16)]
            hist[pl.ds(off + l * NB + g * 16, 16)] = z
        cum = run + plsc.cumsum(tot)
        prev = cum - tot
        m = (prev < q) & (cum >= q)
        bin_ = bin_ + jnp.sum(jnp.where(m, g * 16 + iota, 0))
        base = base + jnp.sum(jnp.where(m, prev, 0))
        return run + jnp.sum(tot), bin_, base

    zi = jnp.int32(0)
    _, bin_, base = lax.fori_loop(0, NB // 16, body, (zi, zi, zi))
    return bin_, base


def _inv_key(key_i32):
    bits = key_i32 ^ jnp.where(key_i32 < 0, INT_MIN, -1)
    return plsc.bitcast(bits, jnp.float32)


def kernel(x):
    mesh = plsc.VectorSubcoreMesh(core_axis_name="c", subcore_axis_name="s")

    @functools.partial(
        pl.kernel,
        out_type=jax.ShapeDtypeStruct((B, N), jnp.float32),
        mesh=mesh,
        compiler_params=pltpu.CompilerParams(needs_layout_passes=False),
        scratch_types=[
            pltpu.VMEM((N,), jnp.float32),
            pltpu.VMEM((N,), jnp.float32),
            pltpu.VMEM((N,), jnp.int32),
            pltpu.SemaphoreType.DMA,
            pltpu.SemaphoreType.DMA,
            pltpu.SemaphoreType.DMA,
            pltpu.SemaphoreType.DMA,
        ],
    )
    def run(x_hbm, out_hbm, buf_a, buf_b, hist, sem_ia, sem_ib, sem_oa, sem_ob):
        wid = lax.axis_index("s") * 2 + lax.axis_index("c")
        lane = lax.iota(jnp.int32, 16)
        ones = jnp.ones((16,), jnp.int32)
        zeros = jnp.zeros((16,), jnp.int32)
        laneoff = lane * NB

        def compute_row(buf):
            @plsc.parallel_loop(0, N, 16, unroll=8)
            def _(i):
                xv = buf[pl.ds(i, 16)]
                bb = plsc.bitcast(xv, jnp.int32)
                t = bb ^ ((bb >> 31) | INT_MIN)
                buf[pl.ds(i, 16)] = plsc.bitcast(t, jnp.float32)
                tu = plsc.bitcast(t, jnp.uint32)
                b1 = (tu >> 24).astype(jnp.int32)
                plsc.addupdate_scatter(hist, [b1 + laneoff], ones)

            b1a, base1a, cnt_a, b1b, cnt_b = _scan_p1(
                hist, jnp.int32(QA), jnp.int32(QA - 1))
            cnt = cnt_a + jnp.where(b1a == b1b, 0, cnt_b)

            def select_in(load_fn, nvec, npad):
                pad_adj = jnp.where(b1a == 0, npad, 0)
                qa_ = (QA - base1a) + pad_adj
                pf = b1a
                for shift in (16, 8, 0):
                    @plsc.parallel_loop(0, nvec, 1, unroll=2)
                    def _(i, _s=shift, _pf=pf):
                        ku = load_fn(i)
                        sub = ((ku >> _s).astype(jnp.int32) & 0xFF) + laneoff
                        pref = (ku >> (_s + 8)).astype(jnp.int32)
                        plsc.addupdate_scatter(
                            hist, [RH + sub], ones, mask=pref == _pf)

                    r, base = _scan1z(hist, qa_, RH)
                    qa_ = qa_ - base
                    pf = pf * 256 + r
                key_a = pf

                kax = key_a ^ INT_MIN
                carry0 = (jnp.full((16,), INT_MIN, jnp.int32), zeros)

                @plsc.parallel_loop(0, nvec, 1, unroll=2, carry=carry0)
                def mc(i, acc):
                    macc, cacc = acc
                    ku = load_fn(i)
                    kx = plsc.bitcast(ku, jnp.int32) ^ INT_MIN
                    m_lt = kx < kax
                    m_cnt = m_lt & ((ku >> 24).astype(jnp.int32) == b1a)
                    macc = jnp.maximum(macc, jnp.where(m_lt, kx, INT_MIN))
                    cacc = cacc + jnp.where(m_cnt, 1, 0)
                    return macc, cacc

                macc, cacc = mc
                low = base1a + jnp.sum(cacc) - pad_adj
                key_b = jnp.where(QA - low >= 2, key_a,
                                  jnp.max(macc) ^ INT_MIN)
                return key_a, key_b

            def small(_):
                @plsc.parallel_loop(0, N, 16, unroll=8, carry=jnp.int32(0))
                def _ext(i, off):
                    tu = plsc.bitcast(buf[pl.ds(i, 16)], jnp.uint32)
                    b1 = (tu >> 24).astype(jnp.int32)
                    m = (b1 == b1a) | (b1 == b1b)
                    plsc.store_compressed(
                        hist.at[pl.ds(off, 16)],
                        plsc.bitcast(tu, jnp.int32), mask=m)
                    return off + jnp.sum(m.astype(jnp.int32))

                nv = (cnt + 15) >> 4
                ka, kb = select_in(
                    lambda i: plsc.bitcast(hist[pl.ds(i * 16, 16)],
                                           jnp.uint32), nv, nv * 16 - cnt)

                @plsc.parallel_loop(0, nv * 16, 16)
                def _(i):
                    hist[pl.ds(i, 16)] = zeros

                return ka, kb

            def big(_):
                return select_in(
                    lambda i: plsc.bitcast(buf[pl.ds(i * 16, 16)],
                                           jnp.uint32),
                    jnp.int32(N // 16), jnp.int32(0))

            key_a, key_b = lax.cond(cnt <= XCAP, small, big, 0)

            va = _inv_key(jnp.full((16,), key_a, jnp.int32))
            vb = _inv_key(jnp.full((16,), key_b, jnp.int32))
            thr = (va + vb) * 0.5
            tb = plsc.bitcast(thr, jnp.int32)
            kthr = plsc.bitcast(tb ^ ((tb >> 31) | INT_MIN), jnp.uint32)

            @plsc.parallel_loop(0, N, 16, unroll=8)
            def _(i):
                tu = plsc.bitcast(buf[pl.ds(i, 16)], jnp.uint32)
                buf[pl.ds(i, 16)] = jnp.where(tu > kthr, 1.0, 0.0)

        r0 = wid * ROWS
        i_a = pltpu.async_copy(x_hbm.at[r0], buf_a, sem_ia)
        i_b = pltpu.async_copy(x_hbm.at[r0 + 1], buf_b, sem_ib)

        @plsc.parallel_loop(0, N, 16, unroll=8)
        def _(i):
            hist[pl.ds(i, 16)] = zeros

        i_a.wait()
        compute_row(buf_a)
        o_a = pltpu.async_copy(buf_a, out_hbm.at[r0], sem_oa)

        i_b.wait()
        o_a.wait()
        i_a = pltpu.async_copy(x_hbm.at[r0 + 2], buf_a, sem_ia)
        compute_row(buf_b)
        o_b = pltpu.async_copy(buf_b, out_hbm.at[r0 + 1], sem_ob)

        i_a.wait()
        o_b.wait()
        i_b = pltpu.async_copy(x_hbm.at[r0 + 3], buf_b, sem_ib)
        compute_row(buf_a)
        o_a = pltpu.async_copy(buf_a, out_hbm.at[r0 + 2], sem_oa)

        i_b.wait()
        compute_row(buf_b)
        o_a.wait()
        o_b = pltpu.async_copy(buf_b, out_hbm.at[r0 + 3], sem_ob)
        o_b.wait()

    return run(x)


# speedup vs baseline: 1.5671x; 1.5671x over previous
import functools

import jax
import jax.numpy as jnp
from jax import lax
from jax.experimental import pallas as pl
from jax.experimental.pallas import tpu as pltpu
from jax.experimental.pallas import tpu_sc as plsc

B = 128
N = 32768
K = 1639
QA = N - K + 1
NB1 = 2048
NBR = 128
NW = 32
ROWS = B // NW
INT_MIN = -(2**31)
XCAP = 16384
RH = 16448


def _scan_p1(hist, qa, qb):
    iota = lax.iota(jnp.int32, 16)
    z = jnp.zeros((16,), jnp.int32)

    def body(g, carry):
        run, bin_a, base_a, cnt_a, bin_b, cnt_b = carry
        tot = hist[pl.ds(g * 16, 16)]
        hist[pl.ds(g * 16, 16)] = z
        for l in range(1, 16):
            tot = tot + hist[pl.ds(l * NB1 + g * 16, 16)]
            hist[pl.ds(l * NB1 + g * 16, 16)] = z
        cum = run + plsc.cumsum(tot)
        prev = cum - tot
        ids = g * 16 + iota
        ma = (prev < qa) & (cum >= qa)
        mb = (prev < qb) & (cum >= qb)
        bin_a = bin_a + jnp.sum(jnp.where(ma, ids, 0))
        base_a = base_a + jnp.sum(jnp.where(ma, prev, 0))
        cnt_a = cnt_a + jnp.sum(jnp.where(ma, tot, 0))
        bin_b = bin_b + jnp.sum(jnp.where(mb, ids, 0))
        cnt_b = cnt_b + jnp.sum(jnp.where(mb, tot, 0))
        return run + jnp.sum(tot), bin_a, base_a, cnt_a, bin_b, cnt_b

    zi = jnp.int32(0)
    _, bin_a, base_a, cnt_a, bin_b, cnt_b = lax.fori_loop(
        0, NB1 // 16, body, (zi, zi, zi, zi, zi, zi))
    return bin_a, base_a, cnt_a, bin_b, cnt_b


def _scan1z(hist, q, off):
    iota = lax.iota(jnp.int32, 16)
    z = jnp.zeros((16,), jnp.int32)

    def body(g, carry):
        run, bin_, base = carry
        tot = hist[pl.ds(off + g * 16, 16)]
        hist[pl.ds(off + g * 16, 16)] = z
        for l in range(1, 16):
            tot = tot + hist[pl.ds(off + l * NBR + g * 16, 16)]
            hist[pl.ds(off + l * NBR + g * 16, 16)] = z
        cum = run + plsc.cumsum(tot)
        prev = cum - tot
        m = (prev < q) & (cum >= q)
        bin_ = bin_ + jnp.sum(jnp.where(m, g * 16 + iota, 0))
        base = base + jnp.sum(jnp.where(m, prev, 0))
        return run + jnp.sum(tot), bin_, base

    zi = jnp.int32(0)
    _, bin_, base = lax.fori_loop(0, NBR // 16, body, (zi, zi, zi))
    return bin_, base


def _inv_key(key_i32):
    bits = key_i32 ^ jnp.where(key_i32 < 0, INT_MIN, -1)
    return plsc.bitcast(bits, jnp.float32)


def kernel(x):
    mesh = plsc.VectorSubcoreMesh(core_axis_name="c", subcore_axis_name="s")

    @functools.partial(
        pl.kernel,
        out_type=jax.ShapeDtypeStruct((B, N), jnp.float32),
        mesh=mesh,
        compiler_params=pltpu.CompilerParams(needs_layout_passes=False),
        scratch_types=[
            pltpu.VMEM((N,), jnp.float32),
            pltpu.VMEM((N,), jnp.float32),
            pltpu.VMEM((N,), jnp.int32),
            pltpu.SemaphoreType.DMA,
            pltpu.SemaphoreType.DMA,
            pltpu.SemaphoreType.DMA,
            pltpu.SemaphoreType.DMA,
        ],
    )
    def run(x_hbm, out_hbm, buf_a, buf_b, hist, sem_ia, sem_ib, sem_oa, sem_ob):
        wid = lax.axis_index("s") * 2 + lax.axis_index("c")
        lane = lax.iota(jnp.int32, 16)
        ones = jnp.ones((16,), jnp.int32)
        zeros = jnp.zeros((16,), jnp.int32)
        laneoff1 = lane * NB1
        laneoff7 = lane * NBR

        def compute_row(buf):
            @plsc.parallel_loop(0, N, 16, unroll=8)
            def _(i):
                xv = buf[pl.ds(i, 16)]
                bb = plsc.bitcast(xv, jnp.int32)
                t = bb ^ ((bb >> 31) | INT_MIN)
                buf[pl.ds(i, 16)] = plsc.bitcast(t, jnp.float32)
                tu = plsc.bitcast(t, jnp.uint32)
                b1 = (tu >> 21).astype(jnp.int32)
                plsc.addupdate_scatter(hist, [b1 + laneoff1], ones)

            b1a, base1a, cnt_a, b1b, cnt_b = _scan_p1(
                hist, jnp.int32(QA), jnp.int32(QA - 1))
            cnt = cnt_a + jnp.where(b1a == b1b, 0, cnt_b)

            def select_in(load_fn, nvec, npad):
                pad_adj = jnp.where(b1a == 0, npad, 0)
                qa_ = (QA - base1a) + pad_adj
                pf = b1a
                for shift in (14, 7, 0):
                    @plsc.parallel_loop(0, nvec, 1, unroll=4)
                    def _(i, _s=shift, _pf=pf):
                        ku = load_fn(i)
                        sub = ((ku >> _s).astype(jnp.int32) & 0x7F) + laneoff7
                        pref = (ku >> (_s + 7)).astype(jnp.int32)
                        plsc.addupdate_scatter(
                            hist, [RH + sub], ones, mask=pref == _pf)

                    r, base = _scan1z(hist, qa_, RH)
                    qa_ = qa_ - base
                    pf = pf * 128 + r
                key_a = pf

                kax = key_a ^ INT_MIN
                carry0 = (jnp.full((16,), INT_MIN, jnp.int32), zeros)

                @plsc.parallel_loop(0, nvec, 1, unroll=4, carry=carry0)
                def mc(i, acc):
                    macc, cacc = acc
                    ku = load_fn(i)
                    kx = plsc.bitcast(ku, jnp.int32) ^ INT_MIN
                    m_lt = kx < kax
                    m_cnt = m_lt & ((ku >> 21).astype(jnp.int32) == b1a)
                    macc = jnp.maximum(macc, jnp.where(m_lt, kx, INT_MIN))
                    cacc = cacc + jnp.where(m_cnt, 1, 0)
                    return macc, cacc

                macc, cacc = mc
                low = base1a + jnp.sum(cacc) - pad_adj
                key_b = jnp.where(QA - low >= 2, key_a,
                                  jnp.max(macc) ^ INT_MIN)
                return key_a, key_b

            def small(_):
                @plsc.parallel_loop(0, N, 16, unroll=8, carry=jnp.int32(0))
                def _ext(i, off):
                    tu = plsc.bitcast(buf[pl.ds(i, 16)], jnp.uint32)
                    b1 = (tu >> 21).astype(jnp.int32)
                    m = (b1 == b1a) | (b1 == b1b)
                    plsc.store_compressed(
                        hist.at[pl.ds(off, 16)],
                        plsc.bitcast(tu, jnp.int32), mask=m)
                    return off + jnp.sum(m.astype(jnp.int32))

                nv = (cnt + 15) >> 4
                ka, kb = select_in(
                    lambda i: plsc.bitcast(hist[pl.ds(i * 16, 16)],
                                           jnp.uint32), nv, nv * 16 - cnt)

                @plsc.parallel_loop(0, nv * 16, 16, unroll=4)
                def _(i):
                    hist[pl.ds(i, 16)] = zeros

                return ka, kb

            def big(_):
                return select_in(
                    lambda i: plsc.bitcast(buf[pl.ds(i * 16, 16)],
                                           jnp.uint32),
                    jnp.int32(N // 16), jnp.int32(0))

            key_a, key_b = lax.cond(cnt <= XCAP, small, big, 0)

            va = _inv_key(jnp.full((16,), key_a, jnp.int32))
            vb = _inv_key(jnp.full((16,), key_b, jnp.int32))
            thr = (va + vb) * 0.5
            tb = plsc.bitcast(thr, jnp.int32)
            kthr = plsc.bitcast(tb ^ ((tb >> 31) | INT_MIN), jnp.uint32)

            @plsc.parallel_loop(0, N, 16, unroll=8)
            def _(i):
                tu = plsc.bitcast(buf[pl.ds(i, 16)], jnp.uint32)
                buf[pl.ds(i, 16)] = jnp.where(tu > kthr, 1.0, 0.0)

        r0 = wid * ROWS
        i_a = pltpu.async_copy(x_hbm.at[r0], buf_a, sem_ia)
        i_b = pltpu.async_copy(x_hbm.at[r0 + 1], buf_b, sem_ib)

        @plsc.parallel_loop(0, N, 16, unroll=8)
        def _(i):
            hist[pl.ds(i, 16)] = zeros

        i_a.wait()
        compute_row(buf_a)
        o_a = pltpu.async_copy(buf_a, out_hbm.at[r0], sem_oa)

        i_b.wait()
        o_a.wait()
        i_a = pltpu.async_copy(x_hbm.at[r0 + 2], buf_a, sem_ia)
        compute_row(buf_b)
        o_b = pltpu.async_copy(buf_b, out_hbm.at[r0 + 1], sem_ob)

        i_a.wait()
        o_b.wait()
        i_b = pltpu.async_copy(x_hbm.at[r0 + 3], buf_b, sem_ib)
        compute_row(buf_a)
        o_a = pltpu.async_copy(buf_a, out_hbm.at[r0 + 2], sem_oa)

        i_b.wait()
        compute_row(buf_b)
        o_a.wait()
        o_b = pltpu.async_copy(buf_b, out_hbm.at[r0 + 3], sem_ob)
        o_b.wait()

    return run(x)


# speedup vs baseline: 1.6247x; 1.0368x over previous
import functools

import jax
import jax.numpy as jnp
from jax import lax
from jax.experimental import pallas as pl
from jax.experimental.pallas import tpu as pltpu
from jax.experimental.pallas import tpu_sc as plsc

B = 128
N = 32768
K = 1639
QA = N - K + 1
NB1 = 2048
NBR = 128
NW = 32
ROWS = B // NW
INT_MIN = -(2**31)
XCAP = 16384
RH = 16448


def _scan_p1(hist, qa, qb):
    iota = lax.iota(jnp.int32, 16)
    z = jnp.zeros((16,), jnp.int32)

    def body(g, carry):
        run, bin_a, base_a, cnt_a, bin_b, cnt_b = carry
        tot = hist[pl.ds(g * 16, 16)]
        hist[pl.ds(g * 16, 16)] = z
        for l in range(1, 16):
            tot = tot + hist[pl.ds(l * NB1 + g * 16, 16)]
            hist[pl.ds(l * NB1 + g * 16, 16)] = z
        cum = run + plsc.cumsum(tot)
        prev = cum - tot
        ids = g * 16 + iota
        ma = (prev < qa) & (cum >= qa)
        mb = (prev < qb) & (cum >= qb)
        bin_a = bin_a + jnp.sum(jnp.where(ma, ids, 0))
        base_a = base_a + jnp.sum(jnp.where(ma, prev, 0))
        cnt_a = cnt_a + jnp.sum(jnp.where(ma, tot, 0))
        bin_b = bin_b + jnp.sum(jnp.where(mb, ids, 0))
        cnt_b = cnt_b + jnp.sum(jnp.where(mb, tot, 0))
        return run + jnp.sum(tot), bin_a, base_a, cnt_a, bin_b, cnt_b

    zi = jnp.int32(0)
    _, bin_a, base_a, cnt_a, bin_b, cnt_b = lax.fori_loop(
        0, NB1 // 16, body, (zi, zi, zi, zi, zi, zi))
    return bin_a, base_a, cnt_a, bin_b, cnt_b


def _scan1z(hist, q, off):
    iota = lax.iota(jnp.int32, 16)
    z = jnp.zeros((16,), jnp.int32)

    def body(g, carry):
        run, bin_, base = carry
        tot = hist[pl.ds(off + g * 16, 16)]
        hist[pl.ds(off + g * 16, 16)] = z
        for l in range(1, 16):
            tot = tot + hist[pl.ds(off + l * NBR + g * 16, 16)]
            hist[pl.ds(off + l * NBR + g * 16, 16)] = z
        cum = run + plsc.cumsum(tot)
        prev = cum - tot
        m = (prev < q) & (cum >= q)
        bin_ = bin_ + jnp.sum(jnp.where(m, g * 16 + iota, 0))
        base = base + jnp.sum(jnp.where(m, prev, 0))
        return run + jnp.sum(tot), bin_, base

    zi = jnp.int32(0)
    _, bin_, base = lax.fori_loop(0, NBR // 16, body, (zi, zi, zi))
    return bin_, base


def _inv_key(key_i32):
    bits = key_i32 ^ jnp.where(key_i32 < 0, INT_MIN, -1)
    return plsc.bitcast(bits, jnp.float32)


def kernel(x):
    mesh = plsc.VectorSubcoreMesh(core_axis_name="c", subcore_axis_name="s")

    @functools.partial(
        pl.kernel,
        out_type=jax.ShapeDtypeStruct((B, N), jnp.float32),
        mesh=mesh,
        compiler_params=pltpu.CompilerParams(needs_layout_passes=False),
        scratch_types=[
            pltpu.VMEM((N,), jnp.float32),
            pltpu.VMEM((N,), jnp.float32),
            pltpu.VMEM((N,), jnp.int32),
            pltpu.SemaphoreType.DMA,
            pltpu.SemaphoreType.DMA,
            pltpu.SemaphoreType.DMA,
            pltpu.SemaphoreType.DMA,
        ],
    )
    def run(x_hbm, out_hbm, buf_a, buf_b, hist, sem_ia, sem_ib, sem_oa, sem_ob):
        wid = lax.axis_index("s") * 2 + lax.axis_index("c")
        lane = lax.iota(jnp.int32, 16)
        ones = jnp.ones((16,), jnp.int32)
        zeros = jnp.zeros((16,), jnp.int32)
        laneoff1 = lane * NB1
        laneoff7 = lane * NBR

        def compute_row(buf):
            @plsc.parallel_loop(0, N, 16, unroll=8)
            def _(i):
                xv = buf[pl.ds(i, 16)]
                bb = plsc.bitcast(xv, jnp.int32)
                t = bb ^ ((bb >> 31) | INT_MIN)
                buf[pl.ds(i, 16)] = plsc.bitcast(t, jnp.float32)
                tu = plsc.bitcast(t, jnp.uint32)
                b1 = (tu >> 21).astype(jnp.int32)
                plsc.addupdate_scatter(hist, [b1 + laneoff1], ones)

            b1a, base1a, cnt_a, b1b, cnt_b = _scan_p1(
                hist, jnp.int32(QA), jnp.int32(QA - 1))
            cnt = cnt_a + jnp.where(b1a == b1b, 0, cnt_b)

            def select_in(load_fn, nvec, npad):
                pad_adj = jnp.where(b1a == 0, npad, 0)
                qa_ = (QA - base1a) + pad_adj
                pf = b1a
                for shift in (14, 7, 0):
                    @plsc.parallel_loop(0, nvec, 1, unroll=4)
                    def _(i, _s=shift, _pf=pf):
                        ku = load_fn(i)
                        sub = ((ku >> _s).astype(jnp.int32) & 0x7F) + laneoff7
                        pref = (ku >> (_s + 7)).astype(jnp.int32)
                        plsc.addupdate_scatter(
                            hist, [RH + sub], ones, mask=pref == _pf)

                    r, base = _scan1z(hist, qa_, RH)
                    qa_ = qa_ - base
                    pf = pf * 128 + r
                key_a = pf

                kax = key_a ^ INT_MIN
                carry0 = (jnp.full((16,), INT_MIN, jnp.int32), zeros)

                @plsc.parallel_loop(0, nvec, 1, unroll=4, carry=carry0)
                def mc(i, acc):
                    macc, cacc = acc
                    ku = load_fn(i)
                    kx = plsc.bitcast(ku, jnp.int32) ^ INT_MIN
                    m_lt = kx < kax
                    m_cnt = m_lt & ((ku >> 21).astype(jnp.int32) == b1a)
                    macc = jnp.maximum(macc, jnp.where(m_lt, kx, INT_MIN))
                    cacc = cacc + jnp.where(m_cnt, 1, 0)
                    return macc, cacc

                macc, cacc = mc
                low = base1a + jnp.sum(cacc) - pad_adj
                key_b = jnp.where(QA - low >= 2, key_a,
                                  jnp.max(macc) ^ INT_MIN)
                return key_a, key_b

            def small(_):
                @plsc.parallel_loop(0, N, 16, unroll=8, carry=zeros)
                def _ext(i, off):
                    tu = plsc.bitcast(buf[pl.ds(i, 16)], jnp.uint32)
                    b1 = (tu >> 21).astype(jnp.int32)
                    m = (b1 == b1a) | (b1 == b1b)
                    mi = jnp.where(m, 1, 0)
                    idx = off + plsc.cumsum(mi) - mi
                    plsc.store_scatter(
                        hist, [idx], plsc.bitcast(tu, jnp.int32), mask=m)
                    return off + plsc.all_reduce_population_count(m)

                nv = (cnt + 15) >> 4
                ka, kb = select_in(
                    lambda i: plsc.bitcast(hist[pl.ds(i * 16, 16)],
                                           jnp.uint32), nv, nv * 16 - cnt)

                @plsc.parallel_loop(0, nv * 16, 16, unroll=4)
                def _(i):
                    hist[pl.ds(i, 16)] = zeros

                return ka, kb

            def big(_):
                return select_in(
                    lambda i: plsc.bitcast(buf[pl.ds(i * 16, 16)],
                                           jnp.uint32),
                    jnp.int32(N // 16), jnp.int32(0))

            key_a, key_b = lax.cond(cnt <= XCAP, small, big, 0)

            va = _inv_key(jnp.full((16,), key_a, jnp.int32))
            vb = _inv_key(jnp.full((16,), key_b, jnp.int32))
            thr = (va + vb) * 0.5
            tb = plsc.bitcast(thr, jnp.int32)
            kthr = plsc.bitcast(tb ^ ((tb >> 31) | INT_MIN), jnp.uint32)

            @plsc.parallel_loop(0, N, 16, unroll=8)
            def _(i):
                tu = plsc.bitcast(buf[pl.ds(i, 16)], jnp.uint32)
                buf[pl.ds(i, 16)] = jnp.where(tu > kthr, 1.0, 0.0)

        r0 = wid * ROWS
        i_a = pltpu.async_copy(x_hbm.at[r0], buf_a, sem_ia)
        i_b = pltpu.async_copy(x_hbm.at[r0 + 1], buf_b, sem_ib)

        @plsc.parallel_loop(0, N, 16, unroll=8)
        def _(i):
            hist[pl.ds(i, 16)] = zeros

        i_a.wait()
        compute_row(buf_a)
        o_a = pltpu.async_copy(buf_a, out_hbm.at[r0], sem_oa)

        i_b.wait()
        o_a.wait()
        i_a = pltpu.async_copy(x_hbm.at[r0 + 2], buf_a, sem_ia)
        compute_row(buf_b)
        o_b = pltpu.async_copy(buf_b, out_hbm.at[r0 + 1], sem_ob)

        i_a.wait()
        o_b.wait()
        i_b = pltpu.async_copy(x_hbm.at[r0 + 3], buf_b, sem_ib)
        compute_row(buf_a)
        o_a = pltpu.async_copy(buf_a, out_hbm.at[r0 + 2], sem_oa)

        i_b.wait()
        compute_row(buf_b)
        o_a.wait()
        o_b = pltpu.async_copy(buf_b, out_hbm.at[r0 + 3], sem_ob)
        o_b.wait()

    return run(x)


# speedup vs baseline: 1.6728x; 1.0296x over previous
import functools

import jax
import jax.numpy as jnp
from jax import lax
from jax.experimental import pallas as pl
from jax.experimental.pallas import tpu as pltpu
from jax.experimental.pallas import tpu_sc as plsc

B = 128
N = 32768
K = 1639
QA = N - K + 1
NB1 = 2048
NBR = 128
NW = 32
ROWS = B // NW
INT_MIN = -(2**31)
XCAP = 16384
RH = 16448


def _scan_p1(hist, qa, qb):
    iota = lax.iota(jnp.int32, 16)
    z = jnp.zeros((16,), jnp.int32)

    def body(g, carry):
        run, bin_a, base_a, cnt_a, bin_b, cnt_b = carry
        tot = hist[pl.ds(g * 16, 16)]
        hist[pl.ds(g * 16, 16)] = z
        for l in range(1, 16):
            tot = tot + hist[pl.ds(l * NB1 + g * 16, 16)]
            hist[pl.ds(l * NB1 + g * 16, 16)] = z
        cum = run + plsc.cumsum(tot)
        prev = cum - tot
        ids = g * 16 + iota
        ma = (prev < qa) & (cum >= qa)
        mb = (prev < qb) & (cum >= qb)
        bin_a = bin_a + jnp.sum(jnp.where(ma, ids, 0))
        base_a = base_a + jnp.sum(jnp.where(ma, prev, 0))
        cnt_a = cnt_a + jnp.sum(jnp.where(ma, tot, 0))
        bin_b = bin_b + jnp.sum(jnp.where(mb, ids, 0))
        cnt_b = cnt_b + jnp.sum(jnp.where(mb, tot, 0))
        return run + jnp.sum(tot), bin_a, base_a, cnt_a, bin_b, cnt_b

    zi = jnp.int32(0)
    _, bin_a, base_a, cnt_a, bin_b, cnt_b = lax.fori_loop(
        0, NB1 // 16, body, (zi, zi, zi, zi, zi, zi))
    return bin_a, base_a, cnt_a, bin_b, cnt_b


def _scan1z(hist, q, off):
    iota = lax.iota(jnp.int32, 16)
    z = jnp.zeros((16,), jnp.int32)

    def body(g, carry):
        run, bin_, base = carry
        tot = hist[pl.ds(off + g * 16, 16)]
        hist[pl.ds(off + g * 16, 16)] = z
        for l in range(1, 16):
            tot = tot + hist[pl.ds(off + l * NBR + g * 16, 16)]
            hist[pl.ds(off + l * NBR + g * 16, 16)] = z
        cum = run + plsc.cumsum(tot)
        prev = cum - tot
        m = (prev < q) & (cum >= q)
        bin_ = bin_ + jnp.sum(jnp.where(m, g * 16 + iota, 0))
        base = base + jnp.sum(jnp.where(m, prev, 0))
        return run + jnp.sum(tot), bin_, base

    zi = jnp.int32(0)
    _, bin_, base = lax.fori_loop(0, NBR // 16, body, (zi, zi, zi))
    return bin_, base


def _inv_key(key_i32):
    bits = key_i32 ^ jnp.where(key_i32 < 0, INT_MIN, -1)
    return plsc.bitcast(bits, jnp.float32)


def kernel(x):
    mesh = plsc.VectorSubcoreMesh(core_axis_name="c", subcore_axis_name="s")

    @functools.partial(
        pl.kernel,
        out_type=jax.ShapeDtypeStruct((B, N), jnp.float32),
        mesh=mesh,
        compiler_params=pltpu.CompilerParams(needs_layout_passes=False),
        scratch_types=[
            pltpu.VMEM((N,), jnp.float32),
            pltpu.VMEM((N,), jnp.float32),
            pltpu.VMEM((N,), jnp.int32),
            pltpu.SemaphoreType.DMA,
            pltpu.SemaphoreType.DMA,
            pltpu.SemaphoreType.DMA,
            pltpu.SemaphoreType.DMA,
        ],
    )
    def run(x_hbm, out_hbm, buf_a, buf_b, hist, sem_ia, sem_ib, sem_oa, sem_ob):
        wid = lax.axis_index("s") * 2 + lax.axis_index("c")
        lane = lax.iota(jnp.int32, 16)
        ones = jnp.ones((16,), jnp.int32)
        zeros = jnp.zeros((16,), jnp.int32)
        laneoff1 = lane * NB1
        laneoff7 = lane * NBR

        def compute_row(buf, prefetch=None):
            @plsc.parallel_loop(0, N, 16, unroll=8)
            def _(i):
                xv = buf[pl.ds(i, 16)]
                bb = plsc.bitcast(xv, jnp.int32)
                t = bb ^ ((bb >> 31) | INT_MIN)
                buf[pl.ds(i, 16)] = plsc.bitcast(t, jnp.float32)
                tu = plsc.bitcast(t, jnp.uint32)
                b1 = (tu >> 21).astype(jnp.int32)
                plsc.addupdate_scatter(hist, [b1 + laneoff1], ones)

            if prefetch is not None:
                prefetch()

            b1a, base1a, cnt_a, b1b, cnt_b = _scan_p1(
                hist, jnp.int32(QA), jnp.int32(QA - 1))
            cnt = cnt_a + jnp.where(b1a == b1b, 0, cnt_b)

            def select_in(load_fn, nvec, npad):
                pad_adj = jnp.where(b1a == 0, npad, 0)
                qa_ = (QA - base1a) + pad_adj
                pf = b1a
                for shift in (14, 7, 0):
                    @plsc.parallel_loop(0, nvec, 1, unroll=4)
                    def _(i, _s=shift, _pf=pf):
                        ku = load_fn(i)
                        sub = ((ku >> _s).astype(jnp.int32) & 0x7F) + laneoff7
                        pref = (ku >> (_s + 7)).astype(jnp.int32)
                        plsc.addupdate_scatter(
                            hist, [RH + sub], ones, mask=pref == _pf)

                    r, base = _scan1z(hist, qa_, RH)
                    qa_ = qa_ - base
                    pf = pf * 128 + r
                key_a = pf

                kax = key_a ^ INT_MIN
                carry0 = (jnp.full((16,), INT_MIN, jnp.int32), zeros)

                @plsc.parallel_loop(0, nvec, 1, unroll=4, carry=carry0)
                def mc(i, acc):
                    macc, cacc = acc
                    ku = load_fn(i)
                    kx = plsc.bitcast(ku, jnp.int32) ^ INT_MIN
                    m_lt = kx < kax
                    m_cnt = m_lt & ((ku >> 21).astype(jnp.int32) == b1a)
                    macc = jnp.maximum(macc, jnp.where(m_lt, kx, INT_MIN))
                    cacc = cacc + jnp.where(m_cnt, 1, 0)
                    return macc, cacc

                macc, cacc = mc
                low = base1a + jnp.sum(cacc) - pad_adj
                key_b = jnp.where(QA - low >= 2, key_a,
                                  jnp.max(macc) ^ INT_MIN)
                return key_a, key_b

            def small(_):
                @plsc.parallel_loop(0, N, 16, unroll=8, carry=zeros)
                def _ext(i, off):
                    tu = plsc.bitcast(buf[pl.ds(i, 16)], jnp.uint32)
                    b1 = (tu >> 21).astype(jnp.int32)
                    m = (b1 == b1a) | (b1 == b1b)
                    mi = jnp.where(m, 1, 0)
                    idx = off + plsc.cumsum(mi) - mi
                    plsc.store_scatter(
                        hist, [idx], plsc.bitcast(tu, jnp.int32), mask=m)
                    return off + plsc.all_reduce_population_count(m)

                nv = (cnt + 15) >> 4
                ka, kb = select_in(
                    lambda i: plsc.bitcast(hist[pl.ds(i * 16, 16)],
                                           jnp.uint32), nv, nv * 16 - cnt)

                @plsc.parallel_loop(0, nv * 16, 16, unroll=4)
                def _(i):
                    hist[pl.ds(i, 16)] = zeros

                return ka, kb

            def big(_):
                return select_in(
                    lambda i: plsc.bitcast(buf[pl.ds(i * 16, 16)],
                                           jnp.uint32),
                    jnp.int32(N // 16), jnp.int32(0))

            key_a, key_b = lax.cond(cnt <= XCAP, small, big, 0)

            va = _inv_key(jnp.full((16,), key_a, jnp.int32))
            vb = _inv_key(jnp.full((16,), key_b, jnp.int32))
            thr = (va + vb) * 0.5
            tb = plsc.bitcast(thr, jnp.int32)
            kthr = plsc.bitcast(tb ^ ((tb >> 31) | INT_MIN), jnp.uint32)

            @plsc.parallel_loop(0, N, 16, unroll=8)
            def _(i):
                tu = plsc.bitcast(buf[pl.ds(i, 16)], jnp.uint32)
                buf[pl.ds(i, 16)] = jnp.where(tu > kthr, 1.0, 0.0)

        r0 = wid * ROWS
        i_a = pltpu.async_copy(x_hbm.at[r0], buf_a, sem_ia)
        i_b = pltpu.async_copy(x_hbm.at[r0 + 1], buf_b, sem_ib)

        @plsc.parallel_loop(0, N, 16, unroll=8)
        def _(i):
            hist[pl.ds(i, 16)] = zeros

        h = {}
        i_a.wait()
        compute_row(buf_a)
        o_a = pltpu.async_copy(buf_a, out_hbm.at[r0], sem_oa)

        i_b.wait()

        def pf1():
            o_a.wait()
            h["ia"] = pltpu.async_copy(x_hbm.at[r0 + 2], buf_a, sem_ia)

        compute_row(buf_b, pf1)
        o_b = pltpu.async_copy(buf_b, out_hbm.at[r0 + 1], sem_ob)

        h["ia"].wait()

        def pf2():
            o_b.wait()
            h["ib"] = pltpu.async_copy(x_hbm.at[r0 + 3], buf_b, sem_ib)

        compute_row(buf_a, pf2)
        o_a = pltpu.async_copy(buf_a, out_hbm.at[r0 + 2], sem_oa)

        h["ib"].wait()
        compute_row(buf_b)
        o_a.wait()
        o_b = pltpu.async_copy(buf_b, out_hbm.at[r0 + 3], sem_ob)
        o_b.wait()

    return run(x)
